# Initial kernel scaffold; baseline (speedup 1.0000x reference)
import jax
import jax.numpy as jnp
from jax.experimental import pallas as pl


def kernel(x, edge_index, W1, b1, W2, b2):
    raise NotImplementedError("write your pallas kernel here")


# speedup vs baseline: 8.3673x; 8.3673x over previous
import functools

import jax
import jax.numpy as jnp
from jax import lax
from jax.experimental import pallas as pl
from jax.experimental.pallas import tpu as pltpu
from jax.experimental.pallas import tpu_sc as plsc

N = 10000
E = 160000
F = 128
NC = 2
NS = 16
EB = 128
NB = E // EB
NDEG = 10240
BN = 2000


def _make_deg():
    mesh = plsc.VectorSubcoreMesh(core_axis_name="c", subcore_axis_name="s")
    per_tile = NDEG // NS

    @functools.partial(
        pl.kernel,
        out_type=jax.ShapeDtypeStruct((NC * NDEG,), jnp.float32),
        mesh=mesh,
        scratch_types=[
            pltpu.VMEM((EB,), jnp.int32),
            pltpu.VMEM((EB,), jnp.float32),
            pltpu.VMEM((per_tile,), jnp.float32),
            pltpu.VMEM_SHARED((NDEG,), jnp.float32),
        ],
    )
    def deg_kernel(dst_hbm, out_hbm, dst_v, ones_v, zeros_v, acc):
        c = lax.axis_index("c")
        s = lax.axis_index("s")
        w = s * NC + c
        for i in range(EB // 16):
            ones_v[pl.ds(i * 16, 16)] = jnp.ones((16,), jnp.float32)
        for i in range(per_tile // 16):
            zeros_v[pl.ds(i * 16, 16)] = jnp.zeros((16,), jnp.float32)
        pltpu.sync_copy(zeros_v, acc.at[pl.ds(s * per_tile, per_tile)])
        plsc.subcore_barrier()
        nb = jnp.where(w < NB % 32, NB // 32 + 1, NB // 32)

        def body(i, carry):
            b = w + i * 32
            pltpu.sync_copy(dst_hbm.at[pl.ds(b * EB, EB)], dst_v)
            pltpu.sync_copy(ones_v, acc.at[dst_v], add=True)
            return carry

        lax.fori_loop(0, nb, body, 0)
        plsc.subcore_barrier()
        pltpu.sync_copy(
            acc.at[pl.ds(s * per_tile, per_tile)],
            out_hbm.at[pl.ds(c * NDEG + s * per_tile, per_tile)],
        )

    return deg_kernel


def _make_prop(C):
    cpc = C // NC
    mesh = plsc.VectorSubcoreMesh(core_axis_name="c", subcore_axis_name="s")
    rpt = 624
    rem = N - NS * rpt

    @functools.partial(
        pl.kernel,
        out_type=jax.ShapeDtypeStruct((C * N, F), jnp.float32),
        mesh=mesh,
        scratch_types=[
            pltpu.VMEM((EB,), jnp.int32),
            pltpu.VMEM((EB,), jnp.int32),
            pltpu.VMEM((EB,), jnp.int32),
            pltpu.VMEM((EB, F), jnp.float32),
            pltpu.VMEM_SHARED((N, F), jnp.float32),
            pltpu.SemaphoreType.DMA,
        ],
    )
    def prop_kernel(hs_hbm, src_hbm, dst_hbm, out_hbm,
                    src_v, adj_v, dst_v, rows_v, acc, sem):
        c = lax.axis_index("c")
        s = lax.axis_index("s")
        nb = jnp.where(s < NB % NS, NB // NS + 1, NB // NS)
        for r in range(cpc):
            chunk = c * cpc + r
            row0 = chunk * N
            pltpu.sync_copy(
                hs_hbm.at[pl.ds(row0 + s * rpt, rpt)],
                acc.at[pl.ds(s * rpt, rpt)],
            )

            @pl.when(s == NS - 1)
            def _():
                pltpu.sync_copy(
                    hs_hbm.at[pl.ds(row0 + NS * rpt, rem)],
                    acc.at[pl.ds(NS * rpt, rem)],
                )

            plsc.subcore_barrier()

            def body(i, carry):
                b = s + i * NS
                base = b * EB
                pltpu.sync_copy(src_hbm.at[pl.ds(base, EB)], src_v)
                pltpu.sync_copy(dst_hbm.at[pl.ds(base, EB)], dst_v)
                off = lax.broadcast(row0, (16,))
                for i2 in range(EB // 16):
                    adj_v[pl.ds(i2 * 16, 16)] = src_v[pl.ds(i2 * 16, 16)] + off
                pltpu.async_copy(hs_hbm.at[adj_v], rows_v, sem).wait()
                pltpu.sync_copy(rows_v, acc.at[dst_v], add=True)
                return carry

            lax.fori_loop(0, nb, body, 0)
            plsc.subcore_barrier()
            pltpu.sync_copy(
                acc.at[pl.ds(s * rpt, rpt)],
                out_hbm.at[pl.ds(row0 + s * rpt, rpt)],
            )

            @pl.when(s == NS - 1)
            def _():
                pltpu.sync_copy(
                    acc.at[pl.ds(NS * rpt, rem)],
                    out_hbm.at[pl.ds(row0 + NS * rpt, rem)],
                )

            if r != cpc - 1:
                plsc.subcore_barrier()

    return prop_kernel


_deg_call = _make_deg()
_prop4_call = _make_prop(4)
_prop2_call = _make_prop(2)


def _dis_body(degp_ref, dis_ref):
    d = 1.0 + degp_ref[0:NDEG // F, :] + degp_ref[NDEG // F:, :]
    dis_ref[...] = lax.rsqrt(d)


def _dis_call(degp):
    out = pl.pallas_call(
        _dis_body,
        out_shape=jax.ShapeDtypeStruct((NDEG // F, F), jnp.float32),
    )(degp.reshape(2 * NDEG // F, F))
    return out.reshape(NDEG)[:N].reshape(N, 1)


def _mm_scale_body(x_ref, w_ref, dis_ref, out_ref):
    h = jnp.dot(x_ref[...], w_ref[...], preferred_element_type=jnp.float32)
    out_ref[...] = h * dis_ref[...]


def _mm_scale_call(x, W, dis2d, C):
    k = x.shape[1]
    return pl.pallas_call(
        _mm_scale_body,
        grid=(N // BN, C),
        in_specs=[
            pl.BlockSpec((BN, k), lambda n, c: (n, 0)),
            pl.BlockSpec((k, F), lambda n, c: (0, c)),
            pl.BlockSpec((BN, 1), lambda n, c: (n, 0)),
        ],
        out_specs=pl.BlockSpec((BN, F), lambda n, c: (c * (N // BN) + n, 0)),
        out_shape=jax.ShapeDtypeStruct((C * N, F), jnp.float32),
    )(x, W, dis2d)


def _mid_body(p_ref, b_ref, w_ref, dis_ref, out_ref, *, nk):
    k = pl.program_id(2)

    @pl.when(k == 0)
    def _():
        out_ref[...] = jnp.zeros_like(out_ref)

    t = jnp.maximum(p_ref[...] * dis_ref[...] + b_ref[0], 0.0)
    out_ref[...] += jnp.dot(t, w_ref[...], preferred_element_type=jnp.float32)

    @pl.when(k == nk - 1)
    def _():
        out_ref[...] *= dis_ref[...]


def _mid_call(p1, b1r, W2, dis2d, C_in, C_out):
    return pl.pallas_call(
        functools.partial(_mid_body, nk=C_in),
        grid=(N // BN, C_out, C_in),
        in_specs=[
            pl.BlockSpec((BN, F), lambda n, f, k: (k * (N // BN) + n, 0)),
            pl.BlockSpec((1, 1, F), lambda n, f, k: (k, 0, 0)),
            pl.BlockSpec((F, F), lambda n, f, k: (k, f)),
            pl.BlockSpec((BN, 1), lambda n, f, k: (n, 0)),
        ],
        out_specs=pl.BlockSpec((BN, F), lambda n, f, k: (f * (N // BN) + n, 0)),
        out_shape=jax.ShapeDtypeStruct((C_out * N, F), jnp.float32),
    )(p1, b1r, W2, dis2d)


def _final_body(p_ref, b_ref, dis_ref, out_ref):
    out_ref[...] = p_ref[...] * dis_ref[...] + b_ref[0]


def _final_call(p2, b2r, dis2d, C):
    return pl.pallas_call(
        _final_body,
        grid=(N // BN, C),
        in_specs=[
            pl.BlockSpec((BN, F), lambda n, f: (f * (N // BN) + n, 0)),
            pl.BlockSpec((1, 1, F), lambda n, f: (f, 0, 0)),
            pl.BlockSpec((BN, 1), lambda n, f: (n, 0)),
        ],
        out_specs=pl.BlockSpec((BN, F), lambda n, f: (n, f)),
        out_shape=jax.ShapeDtypeStruct((N, C * F), jnp.float32),
    )(p2, b2r, dis2d)


def kernel(x, edge_index, W1, b1, W2, b2):
    src = edge_index[0].astype(jnp.int32)
    dst = edge_index[1].astype(jnp.int32)

    degp = _deg_call(dst)
    dis2d = _dis_call(degp)

    hs1 = _mm_scale_call(x, W1, dis2d, 4)
    p1 = _prop4_call(hs1, src, dst)
    hs2 = _mid_call(p1, b1.reshape(4, 1, F), W2, dis2d, 4, 2)
    p2 = _prop2_call(hs2, src, dst)
    z = _final_call(p2, b2.reshape(2, 1, F), dis2d, 2)
    return z
